# baseline (speedup 1.0000x reference)
import functools

import jax
import jax.numpy as jnp
from jax.experimental import pallas as pl


def _body(bits_ref, pat_ref, val_ref, out_ref):
    nb, m, p = pat_ref.shape
    for j in range(nb):
        bits_bf = bits_ref[:, j, :].astype(jnp.bfloat16)
        a = pat_ref[j] > 0.0
        a_bf = a.astype(jnp.bfloat16)
        cross = jax.lax.dot_general(
            bits_bf, a_bf,
            dimension_numbers=(((1,), (1,)), ((), ())),
            preferred_element_type=jnp.float32)
        a_sum = jnp.sum(a.astype(jnp.float32), axis=1)
        score = a_sum[None, :] - 2.0 * cross
        min_s = jnp.min(score, axis=1, keepdims=True)
        m_iota = jax.lax.broadcasted_iota(jnp.int32, score.shape, 1)
        idx = jnp.min(jnp.where(score == min_s, m_iota, m), axis=1)
        onehot = (m_iota == idx[:, None]).astype(jnp.bfloat16)
        v_hard = (val_ref[j] > 0.0).astype(jnp.bfloat16)
        out_ref[j] = jax.lax.dot_general(
            onehot, v_hard,
            dimension_numbers=(((1,), (0,)), ((), ())),
            preferred_element_type=jnp.float32)


@functools.partial(jax.jit, static_argnames=("block_n",))
def _vgram_lookup(bits, pattern_logits, value_logits, block_n=8):
    b, n, p = bits.shape
    _, m, d = value_logits.shape
    grid = (n // block_n,)
    out_t = pl.pallas_call(
        _body,
        grid=grid,
        in_specs=[
            pl.BlockSpec((b, block_n, p), lambda i: (0, i, 0)),
            pl.BlockSpec((block_n, m, p), lambda i: (i, 0, 0)),
            pl.BlockSpec((block_n, m, d), lambda i: (i, 0, 0)),
        ],
        out_specs=pl.BlockSpec((block_n, b, d), lambda i: (i, 0, 0)),
        out_shape=jax.ShapeDtypeStruct((n, b, d), jnp.float32),
    )(bits, pattern_logits, value_logits)
    return out_t.transpose(1, 0, 2)


def kernel(bits, pattern_logits, value_logits):
    return _vgram_lookup(bits, pattern_logits, value_logits)


# speedup vs baseline: 10.2088x; 10.2088x over previous
import functools

import jax
import jax.numpy as jnp
from jax.experimental import pallas as pl


def _body(bits_ref, pat_ref, val_ref, out_ref):
    nb, m, p = pat_ref.shape
    b = out_ref.shape[1]
    for j in range(nb):
        bits_bf = bits_ref[:, j, :].astype(jnp.bfloat16)
        a_bf = (pat_ref[j] > 0.0).astype(jnp.bfloat16)
        cross = jax.lax.dot_general(
            bits_bf, a_bf,
            dimension_numbers=(((1,), (1,)), ((), ())),
            preferred_element_type=jnp.float32)
        a_sum = cross[b:b + 1, :]
        m_iota = jax.lax.broadcasted_iota(jnp.int32, (b, m), 1)
        key = (a_sum - 2.0 * cross[:b, :]) * float(m) \
            + m_iota.astype(jnp.float32)
        min_key = jnp.min(key, axis=1, keepdims=True)
        onehot = (key == min_key).astype(jnp.bfloat16)
        v_hard = (val_ref[j] > 0.0).astype(jnp.bfloat16)
        out_ref[j] = jax.lax.dot_general(
            onehot, v_hard,
            dimension_numbers=(((1,), (0,)), ((), ())),
            preferred_element_type=jnp.float32)


@functools.partial(jax.jit, static_argnames=("block_n",))
def _vgram_lookup(bits, pattern_logits, value_logits, block_n=8):
    b, n, p = bits.shape
    _, m, d = value_logits.shape
    bits_ext = jnp.concatenate(
        [bits, jnp.ones((1, n, p), jnp.float32)], axis=0)
    grid = (n // block_n,)
    out_t = pl.pallas_call(
        _body,
        grid=grid,
        in_specs=[
            pl.BlockSpec((b + 1, block_n, p), lambda i: (0, i, 0)),
            pl.BlockSpec((block_n, m, p), lambda i: (i, 0, 0)),
            pl.BlockSpec((block_n, m, d), lambda i: (i, 0, 0)),
        ],
        out_specs=pl.BlockSpec((block_n, b, d), lambda i: (i, 0, 0)),
        out_shape=jax.ShapeDtypeStruct((n, b, d), jnp.float32),
    )(bits_ext, pattern_logits, value_logits)
    return out_t.transpose(1, 0, 2)


def kernel(bits, pattern_logits, value_logits):
    return _vgram_lookup(bits, pattern_logits, value_logits)


# speedup vs baseline: 11.3806x; 1.1148x over previous
import functools

import jax
import jax.numpy as jnp
from jax.experimental import pallas as pl


def _body(bits_ref, pat_ref, val_ref, out_ref):
    nb, m, p = pat_ref.shape
    b = out_ref.shape[1]
    ones = jnp.ones((8, p), jnp.bfloat16)
    for j in range(nb):
        bits_bf = bits_ref[:, j, :].astype(jnp.bfloat16)
        a_bf = (pat_ref[j] > 0.0).astype(jnp.bfloat16)
        cross = jax.lax.dot_general(
            bits_bf, a_bf,
            dimension_numbers=(((1,), (1,)), ((), ())),
            preferred_element_type=jnp.float32)
        a_sum = jax.lax.dot_general(
            ones, a_bf,
            dimension_numbers=(((1,), (1,)), ((), ())),
            preferred_element_type=jnp.float32)[0:1, :]
        m_iota = jax.lax.broadcasted_iota(jnp.int32, (b, m), 1)
        key = (a_sum - 2.0 * cross) * float(m) \
            + m_iota.astype(jnp.float32)
        min_key = jnp.min(key, axis=1, keepdims=True)
        onehot = (key == min_key).astype(jnp.bfloat16)
        v_hard = (val_ref[j] > 0.0).astype(jnp.bfloat16)
        out_ref[j] = jax.lax.dot_general(
            onehot, v_hard,
            dimension_numbers=(((1,), (0,)), ((), ())),
            preferred_element_type=jnp.float32)


@functools.partial(jax.jit, static_argnames=("block_n",))
def _vgram_lookup(bits, pattern_logits, value_logits, block_n=16):
    b, n, p = bits.shape
    _, m, d = value_logits.shape
    grid = (n // block_n,)
    out_t = pl.pallas_call(
        _body,
        grid=grid,
        in_specs=[
            pl.BlockSpec((b, block_n, p), lambda i: (0, i, 0)),
            pl.BlockSpec((block_n, m, p), lambda i: (i, 0, 0)),
            pl.BlockSpec((block_n, m, d), lambda i: (i, 0, 0)),
        ],
        out_specs=pl.BlockSpec((block_n, b, d), lambda i: (i, 0, 0)),
        out_shape=jax.ShapeDtypeStruct((n, b, d), jnp.float32),
    )(bits, pattern_logits, value_logits)
    return out_t.transpose(1, 0, 2)


def kernel(bits, pattern_logits, value_logits):
    return _vgram_lookup(bits, pattern_logits, value_logits)


# speedup vs baseline: 11.3977x; 1.0015x over previous
import functools

import jax
import jax.numpy as jnp
from jax.experimental import pallas as pl


def _body(bits_ref, pat_ref, val_ref, out_ref):
    nb, m, p = pat_ref.shape
    b = out_ref.shape[1]
    ones = jnp.ones((8, p), jnp.bfloat16)
    for j in range(nb):
        bits_bf = bits_ref[:, j, :].astype(jnp.bfloat16)
        a_bf = (pat_ref[j] > 0.0).astype(jnp.bfloat16)
        cross = jax.lax.dot_general(
            bits_bf, a_bf,
            dimension_numbers=(((1,), (1,)), ((), ())),
            preferred_element_type=jnp.float32)
        a_sum = jax.lax.dot_general(
            ones, a_bf,
            dimension_numbers=(((1,), (1,)), ((), ())),
            preferred_element_type=jnp.float32)[0:1, :]
        m_iota = jax.lax.broadcasted_iota(jnp.int32, (b, m), 1)
        key = (a_sum - 2.0 * cross) * float(m) \
            + m_iota.astype(jnp.float32)
        min_key = jnp.min(key, axis=1, keepdims=True)
        onehot = (key == min_key).astype(jnp.bfloat16)
        v_hard = (val_ref[j] > 0.0).astype(jnp.bfloat16)
        out_ref[j] = jax.lax.dot_general(
            onehot, v_hard,
            dimension_numbers=(((1,), (0,)), ((), ())),
            preferred_element_type=jnp.float32)


@functools.partial(jax.jit, static_argnames=("block_n",))
def _vgram_lookup(bits, pattern_logits, value_logits, block_n=32):
    b, n, p = bits.shape
    _, m, d = value_logits.shape
    grid = (n // block_n,)
    out_t = pl.pallas_call(
        _body,
        grid=grid,
        in_specs=[
            pl.BlockSpec((b, block_n, p), lambda i: (0, i, 0)),
            pl.BlockSpec((block_n, m, p), lambda i: (i, 0, 0)),
            pl.BlockSpec((block_n, m, d), lambda i: (i, 0, 0)),
        ],
        out_specs=pl.BlockSpec((block_n, b, d), lambda i: (i, 0, 0)),
        out_shape=jax.ShapeDtypeStruct((n, b, d), jnp.float32),
    )(bits, pattern_logits, value_logits)
    return out_t.transpose(1, 0, 2)


def kernel(bits, pattern_logits, value_logits):
    return _vgram_lookup(bits, pattern_logits, value_logits)
